# baseline (speedup 1.0000x reference)
import functools

import jax
import jax.numpy as jnp
from jax import lax
from jax.experimental import pallas as pl
from jax.experimental.pallas import tpu as pltpu
from jax.experimental.pallas import tpu_sc as plsc

K_STEPS = 3
ALPHA = 0.5
NS = 16
NC = 2
L = 16
EB = 80
DQ = 64


def _rsqrt16(d):
    half = d * 0.5
    i = plsc.bitcast(d, jnp.int32)
    i = jnp.full((L,), 0x5F3759DF, jnp.int32) - lax.shift_right_arithmetic(
        i, jnp.full((L,), 1, jnp.int32))
    y = plsc.bitcast(i, jnp.float32)
    for _ in range(3):
        y = y * (1.5 - half * y * y)
    return y


def _fill(ref, rows, vec16s, value):
    v = jnp.full((L,), value, jnp.float32)

    @plsc.parallel_loop(0, rows, unroll=4)
    def body(r):
        for j in range(vec16s):
            ref[r, pl.ds(j * L, L)] = v


def _sc_propagate(n, e, nb, nt, nch, ch):
    vq = DQ // L

    mesh = plsc.VectorSubcoreMesh(core_axis_name="c", subcore_axis_name="s")

    @functools.partial(
        pl.kernel,
        mesh=mesh,
        compiler_params=pltpu.CompilerParams(
            use_tc_tiling_on_sc=False, needs_layout_passes=False),
        out_type=(
            jax.ShapeDtypeStruct((n, 4 * DQ), jnp.float32),
            jax.ShapeDtypeStruct((4 * n, DQ), jnp.float32),
        ),
        scratch_types=dict(
            uacc=pltpu.VMEM_SHARED((n, DQ), jnp.float32),
            dga=pltpu.VMEM_SHARED((n, L), jnp.float32),
            srcg0=pltpu.VMEM((nb, EB), jnp.int32),
            dstv=pltpu.VMEM((nb, EB), jnp.int32),
            gbuf0=pltpu.VMEM((EB, DQ), jnp.float32),
            gbuf1=pltpu.VMEM((EB, DQ), jnp.float32),
            gbuf2=pltpu.VMEM((EB, DQ), jnp.float32),
            gbuf3=pltpu.VMEM((EB, DQ), jnp.float32),
            gbuf4=pltpu.VMEM((EB, DQ), jnp.float32),
            ones=pltpu.VMEM((EB, L), jnp.float32),
            nvec=pltpu.VMEM((nt, L), jnp.float32),
            snorm=pltpu.SMEM((nt,), jnp.float32),
            dnorm=pltpu.SMEM((nt,), jnp.float32),
            uch=pltpu.VMEM((ch, DQ), jnp.float32),
            ych=pltpu.VMEM((ch, DQ), jnp.float32),
            ych2=pltpu.VMEM((ch, DQ), jnp.float32),
            gs0=pltpu.SemaphoreType.DMA,
            gs1=pltpu.SemaphoreType.DMA,
            gs2=pltpu.SemaphoreType.DMA,
            gs3=pltpu.SemaphoreType.DMA,
            gs4=pltpu.SemaphoreType.DMA,
            ss0=pltpu.SemaphoreType.DMA,
            ss1=pltpu.SemaphoreType.DMA,
            ss2=pltpu.SemaphoreType.DMA,
            ss3=pltpu.SemaphoreType.DMA,
            ss4=pltpu.SemaphoreType.DMA,
        ),
    )
    def prop(y0_hbm, edges_hbm, out_hbm, h_hbm,
             uacc, dga, srcg0, dstv, gbuf0, gbuf1, gbuf2, gbuf3, gbuf4,
             ones, nvec, snorm, dnorm, uch, ych, ych2,
             gs0, gs1, gs2, gs3, gs4, ss0, ss1, ss2, ss3, ss4):
        c = lax.axis_index("c")
        s = lax.axis_index("s")
        n0 = s * nt

        _fill(ych, ch, vq, 0.0)
        for t in range(nch):
            pltpu.sync_copy(ych, uacc.at[pl.ds(n0 + t * ch, ch)])
        _fill(nvec, nt, 1, 0.0)
        pltpu.sync_copy(nvec, dga.at[pl.ds(n0, nt)])
        _fill(ones, EB, 1, 1.0)
        plsc.subcore_barrier()

        pltpu.sync_copy(edges_hbm.at[0, pl.ds(s * nb, nb)], srcg0)
        pltpu.sync_copy(edges_hbm.at[1, pl.ds(s * nb, nb)], dstv)
        lag = 10

        def deg_scatter(idx):
            def deg_wait():
                pltpu.make_async_copy(ones, dga.at[idx.at[0]], ss0).wait()

            def deg_body(j, _):
                pltpu.async_copy(ones, dga.at[idx.at[j]], ss0, add=True)

                @pl.when(j >= lag)
                def _w():
                    deg_wait()

                return _

            lax.fori_loop(0, nb, deg_body, None)
            for _ in range(lag):
                deg_wait()

        def _shift(delta):
            dv = jnp.full((L,), delta, jnp.int32)

            @plsc.parallel_loop(0, nb, unroll=4)
            def body(j):
                for v in range(EB // L):
                    sl = pl.ds(v * L, L)
                    srcg0[j, sl] = srcg0[j, sl] + dv

        deg_scatter(srcg0)
        _shift(2 * c * n)
        plsc.subcore_barrier()

        def norms_to(dst1d):
            @plsc.parallel_loop(0, nt, unroll=2)
            def norm_body(r):
                y = _rsqrt16(jnp.maximum(nvec[r, pl.ds(0, L)], 1.0))
                dst1d[r] = y[0]

        pltpu.sync_copy(dga.at[pl.ds(n0, nt)], nvec)
        norms_to(snorm)
        _fill(nvec, nt, 1, 0.0)
        pltpu.sync_copy(nvec, dga.at[pl.ds(n0, nt)])
        plsc.subcore_barrier()
        deg_scatter(dstv)
        plsc.subcore_barrier()
        pltpu.sync_copy(dga.at[pl.ds(n0, nt)], nvec)
        norms_to(dnorm)

        ybufs = (ych, ych2)
        ysem = (gs0, gs1)
        stsem = (ss1, ss2)

        def yq_of(i):
            return (2 * c + (i // nch)) * n + n0 + (i % nch) * ch

        pend_st = [None, None]
        pend_y = [None, None]
        pend_y[0] = pltpu.async_copy(
            y0_hbm.at[pl.ds(yq_of(0), ch)], ybufs[0], ysem[0])
        for i in range(2 * nch):
            qq = i % 2
            yb = ybufs[qq]
            ti = i % nch
            pend_y[qq].wait()
            if i + 1 < 2 * nch:
                q2 = (i + 1) % 2
                if pend_st[q2] is not None:
                    pend_st[q2].wait()
                pend_y[q2] = pltpu.async_copy(
                    y0_hbm.at[pl.ds(yq_of(i + 1), ch)], ybufs[q2], ysem[q2])

            @plsc.parallel_loop(0, ch, unroll=4)
            def h0_body(r):
                sn = snorm[ti * ch + r]
                for v in range(vq):
                    sl = pl.ds(v * L, L)
                    yb[r, sl] = yb[r, sl] * sn

            pend_st[qq] = pltpu.async_copy(
                yb, h_hbm.at[pl.ds(yq_of(i), ch)], stsem[qq])
        for qq in range(2):
            if pend_st[qq] is not None:
                pend_st[qq].wait()
        plsc.subcore_barrier()

        for k in range(K_STEPS):
            last = k == K_STEPS - 1
            for p in range(2):
                if p == 1:
                    _shift(n)

                nbuf = 5
                bufs = (gbuf0, gbuf1, gbuf2, gbuf3, gbuf4)
                gss = (gs0, gs1, gs2, gs3, gs4)
                sss = (ss0, ss1, ss2, ss3, ss4)

                def gather(j, q):
                    pltpu.async_copy(h_hbm.at[srcg0.at[j]], bufs[q], gss[q])

                def gwait(q):
                    pltpu.make_async_copy(
                        h_hbm.at[srcg0.at[0]], bufs[q], gss[q]).wait()

                def scat(j, q):
                    pltpu.async_copy(
                        bufs[q], uacc.at[dstv.at[j]], sss[q], add=True)

                def swait(q):
                    pltpu.make_async_copy(
                        bufs[q], uacc.at[dstv.at[0]], sss[q]).wait()

                for q in range(nbuf):
                    gather(q, q)

                def edge_quad(i, _):
                    j = nbuf * i
                    for q in range(nbuf):
                        gwait(q)
                        scat(j + q, q)
                    for q in range(nbuf):
                        swait(q)

                        @pl.when(j + nbuf + q < nb)
                        def _g(jq=j + nbuf + q, q=q):
                            gather(jq, q)

                    return _

                lax.fori_loop(0, nb // nbuf, edge_quad, None)
                for q in range(nb - nbuf * (nb // nbuf)):
                    gwait(q)
                    scat(nbuf * (nb // nbuf) + q, q)
                    swait(q)
                if p == 1:
                    _shift(-n)
                plsc.subcore_barrier()

                yq0 = (2 * c + p) * n + n0
                pend_h = None
                pend_z = [None, None]
                pend_yd = [None, None]
                pend_yd[0] = pltpu.async_copy(
                    y0_hbm.at[pl.ds(yq0, ch)], ybufs[0], ysem[0])
                for t in range(nch):
                    g0 = n0 + t * ch
                    if pend_h is not None:
                        pend_h.wait()
                    pltpu.sync_copy(uacc.at[pl.ds(g0, ch)], uch)
                    pend_yd[t % 2].wait()
                    if t + 1 < nch:
                        q2 = (t + 1) % 2
                        if pend_z[q2] is not None:
                            pend_z[q2].wait()
                            pend_z[q2] = None
                        pend_yd[q2] = pltpu.async_copy(
                            y0_hbm.at[pl.ds(yq0 + (t + 1) * ch, ch)],
                            ybufs[q2], ysem[q2])
                    yb = ybufs[t % 2]

                    @plsc.parallel_loop(0, ch, unroll=4)
                    def dense_body(r):
                        sn = snorm[t * ch + r]
                        dn = dnorm[t * ch + r]
                        if last:
                            a = (1.0 - ALPHA) * dn
                            b = jnp.float32(ALPHA)
                        else:
                            a = (1.0 - ALPHA) * sn * dn
                            b = ALPHA * sn
                        for v in range(vq):
                            sl = pl.ds(v * L, L)
                            uch[r, sl] = a * uch[r, sl] + b * yb[r, sl]
                    if last:
                        pend_h = pltpu.async_copy(
                            uch,
                            out_hbm.at[pl.ds(g0, ch),
                                       pl.ds((2 * c + p) * DQ, DQ)], ss0)
                    else:
                        pend_h = pltpu.async_copy(
                            uch, h_hbm.at[pl.ds(yq0 + t * ch, ch)], ss0)
                    if not (last and p == 1):
                        _fill(yb, ch, vq, 0.0)
                        pend_z[t % 2] = pltpu.async_copy(
                            yb, uacc.at[pl.ds(g0, ch)], stsem[t % 2])
                pend_h.wait()
                for q2 in range(2):
                    if pend_z[q2] is not None:
                        pend_z[q2].wait()
                if not (last and p == 1):
                    plsc.subcore_barrier()

    return prop


def _tc_matmul(n, d, bn):
    def body(x_ref, w_ref, o_ref):
        o_ref[...] = lax.dot_general(
            x_ref[...], w_ref[...], (((1,), (1,)), ((), ())),
            preferred_element_type=jnp.float32)

    nblk = n // bn
    return pl.pallas_call(
        body,
        grid=(4, nblk),
        in_specs=[
            pl.BlockSpec((bn, d), lambda q, i: (i, 0)),
            pl.BlockSpec((DQ, d), lambda q, i: (q, 0)),
        ],
        out_specs=pl.BlockSpec((bn, DQ), lambda q, i: (q * nblk + i, 0)),
        out_shape=jax.ShapeDtypeStruct((4 * n, DQ), jnp.float32),
    )


def kernel(x, edge_index, W):
    n, d = x.shape
    e = edge_index.shape[1]

    nb = e // (NS * EB)
    edges = edge_index.astype(jnp.int32).reshape(2, NS * nb, EB)

    nt = n // NS
    ch = 125
    nch = nt // ch

    y0 = _tc_matmul(n, d, bn=2000)(x, W)
    out, _ = _sc_propagate(n, e, nb, nt, nch, ch)(y0, edges)
    return out
